# baseline (speedup 1.0000x reference)
import functools

import jax
import jax.numpy as jnp
from jax import lax
from jax.experimental import pallas as pl
from jax.experimental.pallas import tpu as pltpu
from jax.experimental.pallas import tpu_sc as plsc

N_NODES = 100000
DIM = 128
N_EDGES = 1600000
N_PAD = 100352
SLICE = N_PAD // 16
NCHUNK = SLICE // 16
N_CORES = 2
N_SUB = 16
EPC = N_EDGES // N_CORES
EPT = EPC // N_SUB
EW = 10000
NWIN = EPT // EW

_mesh = plsc.VectorSubcoreMesh(core_axis_name="c", subcore_axis_name="s")


def _esplit_body(e_ref, src_ref, dst_ref):
    src_ref[...] = e_ref[0, :]
    dst_ref[...] = e_ref[1, :]


def _esplit(edge_index):
    eb = 102400
    return pl.pallas_call(
        _esplit_body,
        grid=((N_EDGES + eb - 1) // eb,),
        in_specs=[pl.BlockSpec((2, eb), lambda i: (0, i))],
        out_specs=[
            pl.BlockSpec((eb,), lambda i: (i,)),
            pl.BlockSpec((eb,), lambda i: (i,)),
        ],
        out_shape=[jax.ShapeDtypeStruct((N_EDGES,), jnp.int32)] * 2,
    )(edge_index)


def _proj_body(x_ref, w_ref, p_ref):
    y = lax.dot_general(w_ref[...], x_ref[...], (((1,), (1,)), ((), ())),
                        preferred_element_type=jnp.float32)
    p_ref[...] = y.reshape(-1, DIM)


def _proj(x, w1row):
    br = 2048
    rows = N_PAD // DIM
    return pl.pallas_call(
        _proj_body,
        grid=(N_PAD // br,),
        in_specs=[
            pl.BlockSpec((br, DIM), lambda i: (i, 0)),
            pl.BlockSpec((1, DIM), lambda i: (0, 0)),
        ],
        out_specs=pl.BlockSpec((br // DIM, DIM), lambda i: (i, 0)),
        out_shape=jax.ShapeDtypeStruct((rows, DIM), jnp.float32),
    )(x, w1row)


def _mp_body(stage_mode, *refs):
    if stage_mode == 0:
        (p0, src_h, dst_h, pa_o, pb_o, *rest) = refs
    else:
        (pa_i, pb_i, hp_i, cvec, src_h, dst_h, pa_o, pb_o, h_o, *rest) = refs
    (h_sh, acc_sh, buf_a, buf_b, buf_c, hbuf, cbuf,
     srcb0, srcb1, dstb0, dstb1, dstb2, valb0, valb1, valb2,
     sem_s0, sem_s1, sem_d0, sem_d1, sem_v0, sem_v1, sem_v2) = rest
    srcb = (srcb0, srcb1)
    dstb = (dstb0, dstb1, dstb2)
    valb = (valb0, valb1, valb2)
    sem_s = (sem_s0, sem_s1)
    sem_d = (sem_d0, sem_d1)
    sem_v = (sem_v0, sem_v1, sem_v2)
    c = lax.axis_index("c")
    s = lax.axis_index("s")
    sl = pl.ds(s * SLICE, SLICE)

    if stage_mode == 0:
        pltpu.sync_copy(p0.at[sl], hbuf)
    else:
        pltpu.sync_copy(pa_i.at[sl], buf_a)
        pltpu.sync_copy(pb_i.at[sl], buf_b)
        pltpu.sync_copy(hp_i.at[sl], buf_c)
        pltpu.sync_copy(cvec, cbuf)
        wn = cbuf[pl.ds(0, 16)]
        bn = cbuf[pl.ds(16, 16)]
        ws = cbuf[pl.ds(32, 16)]

        def ew_step(i, carry):
            d = pl.ds(i * 16, 16)
            hbuf[d] = jnp.maximum(
                wn * (buf_a[d] + buf_b[d]) + bn + ws * buf_c[d], 0.0)
            return carry

        lax.fori_loop(0, NCHUNK, ew_step, 0)

    pltpu.sync_copy(hbuf, h_sh.at[sl])
    if stage_mode != 0:
        @pl.when(c == 0)
        def _():
            pltpu.sync_copy(hbuf, h_o.at[sl])

    def z_step(i, carry):
        buf_a[pl.ds(i * 16, 16)] = jnp.zeros((16,), jnp.float32)
        return carry

    lax.fori_loop(0, NCHUNK, z_step, 0)
    pltpu.sync_copy(buf_a, acc_sh.at[sl])
    plsc.subcore_barrier()

    ebase = c * EPC + s * EPT

    def idx_start(k):
        wb = ebase + k * EW
        return (
            pltpu.async_copy(src_h.at[pl.ds(wb, EW)], srcb[k % 2], sem_s[k % 2]),
            pltpu.async_copy(dst_h.at[pl.ds(wb, EW)], dstb[k % 3], sem_d[k % 2]),
        )

    idx_d = [None] * (NWIN + 1)
    scat = [None] * NWIN
    idx_d[0] = idx_start(0)
    for k in range(NWIN):
        if k >= 2:
            scat[k - 2].wait()
        idx_d[k][0].wait()
        idx_d[k][1].wait()
        if k + 1 < NWIN:
            idx_d[k + 1] = idx_start(k + 1)
        pltpu.sync_copy(h_sh.at[srcb[k % 2]], valb[k % 3])
        scat[k] = pltpu.async_copy(
            valb[k % 3], acc_sh.at[dstb[k % 3]], sem_v[k % 3], add=True)
    for k in range(max(NWIN - 2, 0), NWIN):
        scat[k].wait()
    plsc.subcore_barrier()

    pltpu.sync_copy(acc_sh.at[sl], hbuf)

    @pl.when(c == 0)
    def _():
        pltpu.sync_copy(hbuf, pa_o.at[sl])

    @pl.when(c == 1)
    def _():
        pltpu.sync_copy(hbuf, pb_o.at[sl])


_SC_SCRATCH = [
    pltpu.VMEM_SHARED((N_PAD,), jnp.float32),
    pltpu.VMEM_SHARED((N_PAD,), jnp.float32),
    pltpu.VMEM((SLICE,), jnp.float32),
    pltpu.VMEM((SLICE,), jnp.float32),
    pltpu.VMEM((SLICE,), jnp.float32),
    pltpu.VMEM((SLICE,), jnp.float32),
    pltpu.VMEM((48,), jnp.float32),
    pltpu.VMEM((EW,), jnp.int32),
    pltpu.VMEM((EW,), jnp.int32),
    pltpu.VMEM((EW,), jnp.int32),
    pltpu.VMEM((EW,), jnp.int32),
    pltpu.VMEM((EW,), jnp.int32),
    pltpu.VMEM((EW,), jnp.float32),
    pltpu.VMEM((EW,), jnp.float32),
    pltpu.VMEM((EW,), jnp.float32),
    pltpu.SemaphoreType.DMA,
    pltpu.SemaphoreType.DMA,
    pltpu.SemaphoreType.DMA,
    pltpu.SemaphoreType.DMA,
    pltpu.SemaphoreType.DMA,
    pltpu.SemaphoreType.DMA,
    pltpu.SemaphoreType.DMA,
]

_vec = jax.ShapeDtypeStruct((N_PAD,), jnp.float32)

_layer0 = pl.kernel(
    functools.partial(_mp_body, 0),
    out_type=[_vec, _vec],
    mesh=_mesh,
    scratch_types=_SC_SCRATCH,
)

_layer12 = pl.kernel(
    functools.partial(_mp_body, 1),
    out_type=[_vec, _vec, _vec],
    mesh=_mesh,
    scratch_types=_SC_SCRATCH,
)


_BRF = 112
_NGF = (N_PAD // DIM) // _BRF


def _final_body(pa_ref, pb_ref, h2_ref, wn2_ref, bn2_ref, ws2_ref,
                w1_ref, b1_ref, w2_ref, b2_ref, out_ref, acc_ref):
    g = pl.program_id(0)

    @pl.when(g == 0)
    def _():
        acc_ref[0] = 0.0

    wn2 = wn2_ref[0, 0]
    bn2 = bn2_ref[0, 0]
    ws2 = ws2_ref[0, 0]
    a = pa_ref[...] + pb_ref[...]
    h3 = jnp.maximum(wn2 * a + bn2 + ws2 * h2_ref[...], 0.0)
    rid = lax.broadcasted_iota(jnp.int32, (_BRF, DIM), 0) + g * _BRF
    lid = lax.broadcasted_iota(jnp.int32, (_BRF, DIM), 1)
    h3 = jnp.where(rid * DIM + lid < N_NODES, h3, 0.0)
    acc_ref[0] += jnp.sum(h3)

    @pl.when(g == _NGF - 1)
    def _():
        hg = acc_ref[0]
        z = (hg * w1_ref[...] + b1_ref[...]) * 1000.0
        sg = 1.0 / (1.0 + jnp.exp(-z))
        o = jnp.sum(sg * w2_ref[...], axis=1).reshape(1, 4) + b2_ref[...]
        o = jnp.maximum(o, 0.0)
        m = jnp.max(o)
        lse = m + jnp.log(jnp.sum(jnp.exp(o - m)))
        out_ref[...] = o - lse


def _final(pa, pb, h2, wn2, bn2, ws2, w1, b1, w2, b2):
    rows = N_PAD // DIM
    return pl.pallas_call(
        _final_body,
        grid=(_NGF,),
        in_specs=[
            pl.BlockSpec((_BRF, DIM), lambda g: (g, 0)),
            pl.BlockSpec((_BRF, DIM), lambda g: (g, 0)),
            pl.BlockSpec((_BRF, DIM), lambda g: (g, 0)),
            pl.BlockSpec((1, 1), lambda g: (0, 0)),
            pl.BlockSpec((1, 1), lambda g: (0, 0)),
            pl.BlockSpec((1, 1), lambda g: (0, 0)),
            pl.BlockSpec((1, 8), lambda g: (0, 0)),
            pl.BlockSpec((1, 8), lambda g: (0, 0)),
            pl.BlockSpec((4, 8), lambda g: (0, 0)),
            pl.BlockSpec((1, 4), lambda g: (0, 0)),
        ],
        out_specs=pl.BlockSpec((1, 4), lambda g: (0, 0)),
        out_shape=jax.ShapeDtypeStruct((1, 4), jnp.float32),
        scratch_shapes=[pltpu.SMEM((1,), jnp.float32)],
    )(pa.reshape(rows, DIM), pb.reshape(rows, DIM), h2.reshape(rows, DIM),
      wn2, bn2, ws2, w1, b1, w2, b2)


def kernel(x, edge_index, Wn0, bn0, Ws0, Wn1, bn1, Ws1, Wn2, bn2, Ws2,
           W1, b1, W2, b2):
    p0 = _proj(x, Wn0.reshape(1, DIM)).reshape(N_PAD)
    src, dst = _esplit(edge_index)
    ones = jnp.ones((16,), jnp.float32)
    c1 = jnp.concatenate([ones, jnp.broadcast_to(bn0[0], (16,)), ones])
    c2 = jnp.concatenate([jnp.broadcast_to(Wn1[0, 0], (16,)),
                          jnp.broadcast_to(bn1[0], (16,)),
                          jnp.broadcast_to(Ws1[0, 0], (16,))])
    pa0, pb0 = _layer0(p0, src, dst)
    s0 = _proj(x, Ws0.reshape(1, DIM)).reshape(N_PAD)
    pa1, pb1, h1 = _layer12(pa0, pb0, s0, c1, src, dst)
    pa2, pb2, h2 = _layer12(pa1, pb1, h1, c2, src, dst)
    return _final(pa2, pb2, h2,
                  Wn2.reshape(1, 1), bn2.reshape(1, 1), Ws2.reshape(1, 1),
                  W1.reshape(1, 8), b1.reshape(1, 8), W2, b2.reshape(1, 4))


# speedup vs baseline: 1.0462x; 1.0462x over previous
import functools

import jax
import jax.numpy as jnp
from jax import lax
from jax.experimental import pallas as pl
from jax.experimental.pallas import tpu as pltpu
from jax.experimental.pallas import tpu_sc as plsc

N_NODES = 100000
DIM = 128
N_EDGES = 1600000
N_PAD = 100352
SLICE = N_PAD // 16
NCHUNK = SLICE // 16
N_CORES = 2
N_SUB = 16
EPC = N_EDGES // N_CORES
EPT = EPC // N_SUB
EW = 10000
NWIN = EPT // EW

_mesh = plsc.VectorSubcoreMesh(core_axis_name="c", subcore_axis_name="s")


def _esplit_body(e_ref, src_ref, dst_ref):
    src_ref[...] = e_ref[0, :]
    dst_ref[...] = e_ref[1, :]


def _esplit(edge_index):
    eb = 204800
    return pl.pallas_call(
        _esplit_body,
        grid=((N_EDGES + eb - 1) // eb,),
        in_specs=[pl.BlockSpec((2, eb), lambda i: (0, i))],
        out_specs=[
            pl.BlockSpec((eb,), lambda i: (i,)),
            pl.BlockSpec((eb,), lambda i: (i,)),
        ],
        out_shape=[jax.ShapeDtypeStruct((N_EDGES,), jnp.int32)] * 2,
    )(edge_index)


def _proj_body(x_ref, w_ref, p_ref, s_ref):
    y = lax.dot_general(w_ref[...], x_ref[...], (((1,), (1,)), ((), ())),
                        preferred_element_type=jnp.float32)
    p_ref[...] = y[0].reshape(-1, DIM)
    s_ref[...] = y[1].reshape(-1, DIM)


def _proj(x, w2):
    br = 2048
    rows = N_PAD // DIM
    return pl.pallas_call(
        _proj_body,
        grid=(N_PAD // br,),
        in_specs=[
            pl.BlockSpec((br, DIM), lambda i: (i, 0)),
            pl.BlockSpec((2, DIM), lambda i: (0, 0)),
        ],
        out_specs=[
            pl.BlockSpec((br // DIM, DIM), lambda i: (i, 0)),
            pl.BlockSpec((br // DIM, DIM), lambda i: (i, 0)),
        ],
        out_shape=[jax.ShapeDtypeStruct((rows, DIM), jnp.float32)] * 2,
    )(x, w2)


def _mp_body(stage_mode, *refs):
    if stage_mode == 0:
        (p0, src_h, dst_h, pa_o, pb_o, *rest) = refs
    else:
        (pa_i, pb_i, hp_i, cvec, src_h, dst_h, pa_o, pb_o, h_o, *rest) = refs
    (h_sh, acc_sh, buf_a, buf_b, buf_c, hbuf, cbuf,
     srcb0, srcb1, dstb0, dstb1, dstb2, valb0, valb1, valb2,
     sem_s0, sem_s1, sem_d0, sem_d1, sem_v0, sem_v1, sem_v2) = rest
    srcb = (srcb0, srcb1)
    dstb = (dstb0, dstb1, dstb2)
    valb = (valb0, valb1, valb2)
    sem_s = (sem_s0, sem_s1)
    sem_d = (sem_d0, sem_d1)
    sem_v = (sem_v0, sem_v1, sem_v2)
    c = lax.axis_index("c")
    s = lax.axis_index("s")
    sl = pl.ds(s * SLICE, SLICE)

    if stage_mode == 0:
        pltpu.sync_copy(p0.at[sl], hbuf)
    else:
        pltpu.sync_copy(pa_i.at[sl], buf_a)
        pltpu.sync_copy(pb_i.at[sl], buf_b)
        pltpu.sync_copy(hp_i.at[sl], buf_c)
        pltpu.sync_copy(cvec, cbuf)
        wn = cbuf[pl.ds(0, 16)]
        bn = cbuf[pl.ds(16, 16)]
        ws = cbuf[pl.ds(32, 16)]

        def ew_step(i, carry):
            d = pl.ds(i * 16, 16)
            hbuf[d] = jnp.maximum(
                wn * (buf_a[d] + buf_b[d]) + bn + ws * buf_c[d], 0.0)
            return carry

        lax.fori_loop(0, NCHUNK, ew_step, 0)

    pltpu.sync_copy(hbuf, h_sh.at[sl])
    if stage_mode != 0:
        @pl.when(c == 0)
        def _():
            pltpu.sync_copy(hbuf, h_o.at[sl])

    def z_step(i, carry):
        buf_a[pl.ds(i * 16, 16)] = jnp.zeros((16,), jnp.float32)
        return carry

    lax.fori_loop(0, NCHUNK, z_step, 0)
    pltpu.sync_copy(buf_a, acc_sh.at[sl])
    plsc.subcore_barrier()

    ebase = c * EPC + s * EPT

    def idx_start(k):
        wb = ebase + k * EW
        return (
            pltpu.async_copy(src_h.at[pl.ds(wb, EW)], srcb[k % 2], sem_s[k % 2]),
            pltpu.async_copy(dst_h.at[pl.ds(wb, EW)], dstb[k % 3], sem_d[k % 2]),
        )

    idx_d = [None] * (NWIN + 1)
    scat = [None] * NWIN
    idx_d[0] = idx_start(0)
    for k in range(NWIN):
        if k >= 2:
            scat[k - 2].wait()
        idx_d[k][0].wait()
        idx_d[k][1].wait()
        if k + 1 < NWIN:
            idx_d[k + 1] = idx_start(k + 1)
        pltpu.sync_copy(h_sh.at[srcb[k % 2]], valb[k % 3])
        scat[k] = pltpu.async_copy(
            valb[k % 3], acc_sh.at[dstb[k % 3]], sem_v[k % 3], add=True)
    for k in range(max(NWIN - 2, 0), NWIN):
        scat[k].wait()
    plsc.subcore_barrier()

    pltpu.sync_copy(acc_sh.at[sl], hbuf)

    @pl.when(c == 0)
    def _():
        pltpu.sync_copy(hbuf, pa_o.at[sl])

    @pl.when(c == 1)
    def _():
        pltpu.sync_copy(hbuf, pb_o.at[sl])


_SC_SCRATCH = [
    pltpu.VMEM_SHARED((N_PAD,), jnp.float32),
    pltpu.VMEM_SHARED((N_PAD,), jnp.float32),
    pltpu.VMEM((SLICE,), jnp.float32),
    pltpu.VMEM((SLICE,), jnp.float32),
    pltpu.VMEM((SLICE,), jnp.float32),
    pltpu.VMEM((SLICE,), jnp.float32),
    pltpu.VMEM((48,), jnp.float32),
    pltpu.VMEM((EW,), jnp.int32),
    pltpu.VMEM((EW,), jnp.int32),
    pltpu.VMEM((EW,), jnp.int32),
    pltpu.VMEM((EW,), jnp.int32),
    pltpu.VMEM((EW,), jnp.int32),
    pltpu.VMEM((EW,), jnp.float32),
    pltpu.VMEM((EW,), jnp.float32),
    pltpu.VMEM((EW,), jnp.float32),
    pltpu.SemaphoreType.DMA,
    pltpu.SemaphoreType.DMA,
    pltpu.SemaphoreType.DMA,
    pltpu.SemaphoreType.DMA,
    pltpu.SemaphoreType.DMA,
    pltpu.SemaphoreType.DMA,
    pltpu.SemaphoreType.DMA,
]

_vec = jax.ShapeDtypeStruct((N_PAD,), jnp.float32)

_layer0 = pl.kernel(
    functools.partial(_mp_body, 0),
    out_type=[_vec, _vec],
    mesh=_mesh,
    scratch_types=_SC_SCRATCH,
)

_layer12 = pl.kernel(
    functools.partial(_mp_body, 1),
    out_type=[_vec, _vec, _vec],
    mesh=_mesh,
    scratch_types=_SC_SCRATCH,
)


_BRF = 784
_NGF = (N_PAD // DIM) // _BRF


def _final_body(pa_ref, pb_ref, h2_ref, wn2_ref, bn2_ref, ws2_ref,
                w1_ref, b1_ref, w2_ref, b2_ref, out_ref, acc_ref):
    g = pl.program_id(0)

    @pl.when(g == 0)
    def _():
        acc_ref[0] = 0.0

    wn2 = wn2_ref[0, 0]
    bn2 = bn2_ref[0, 0]
    ws2 = ws2_ref[0, 0]
    a = pa_ref[...] + pb_ref[...]
    h3 = jnp.maximum(wn2 * a + bn2 + ws2 * h2_ref[...], 0.0)
    rid = lax.broadcasted_iota(jnp.int32, (_BRF, DIM), 0) + g * _BRF
    lid = lax.broadcasted_iota(jnp.int32, (_BRF, DIM), 1)
    h3 = jnp.where(rid * DIM + lid < N_NODES, h3, 0.0)
    acc_ref[0] += jnp.sum(h3)

    @pl.when(g == _NGF - 1)
    def _():
        hg = acc_ref[0]
        z = (hg * w1_ref[...] + b1_ref[...]) * 1000.0
        sg = 1.0 / (1.0 + jnp.exp(-z))
        o = jnp.sum(sg * w2_ref[...], axis=1).reshape(1, 4) + b2_ref[...]
        o = jnp.maximum(o, 0.0)
        m = jnp.max(o)
        lse = m + jnp.log(jnp.sum(jnp.exp(o - m)))
        out_ref[...] = o - lse


def _final(pa, pb, h2, wn2, bn2, ws2, w1, b1, w2, b2):
    rows = N_PAD // DIM
    return pl.pallas_call(
        _final_body,
        grid=(_NGF,),
        in_specs=[
            pl.BlockSpec((_BRF, DIM), lambda g: (g, 0)),
            pl.BlockSpec((_BRF, DIM), lambda g: (g, 0)),
            pl.BlockSpec((_BRF, DIM), lambda g: (g, 0)),
            pl.BlockSpec((1, 1), lambda g: (0, 0)),
            pl.BlockSpec((1, 1), lambda g: (0, 0)),
            pl.BlockSpec((1, 1), lambda g: (0, 0)),
            pl.BlockSpec((1, 8), lambda g: (0, 0)),
            pl.BlockSpec((1, 8), lambda g: (0, 0)),
            pl.BlockSpec((4, 8), lambda g: (0, 0)),
            pl.BlockSpec((1, 4), lambda g: (0, 0)),
        ],
        out_specs=pl.BlockSpec((1, 4), lambda g: (0, 0)),
        out_shape=jax.ShapeDtypeStruct((1, 4), jnp.float32),
        scratch_shapes=[pltpu.SMEM((1,), jnp.float32)],
    )(pa.reshape(rows, DIM), pb.reshape(rows, DIM), h2.reshape(rows, DIM),
      wn2, bn2, ws2, w1, b1, w2, b2)


def kernel(x, edge_index, Wn0, bn0, Ws0, Wn1, bn1, Ws1, Wn2, bn2, Ws2,
           W1, b1, W2, b2):
    w2 = jnp.concatenate([Wn0.reshape(1, DIM), Ws0.reshape(1, DIM)], axis=0)
    p0_2d, s0_2d = _proj(x, w2)
    p0 = p0_2d.reshape(N_PAD)
    s0 = s0_2d.reshape(N_PAD)
    src, dst = _esplit(edge_index)
    ones = jnp.ones((16,), jnp.float32)
    c1 = jnp.concatenate([ones, jnp.broadcast_to(bn0[0], (16,)), ones])
    c2 = jnp.concatenate([jnp.broadcast_to(Wn1[0, 0], (16,)),
                          jnp.broadcast_to(bn1[0], (16,)),
                          jnp.broadcast_to(Ws1[0, 0], (16,))])
    pa0, pb0 = _layer0(p0, src, dst)
    pa1, pb1, h1 = _layer12(pa0, pb0, s0, c1, src, dst)
    pa2, pb2, h2 = _layer12(pa1, pb1, h1, c2, src, dst)
    return _final(pa2, pb2, h2,
                  Wn2.reshape(1, 1), bn2.reshape(1, 1), Ws2.reshape(1, 1),
                  W1.reshape(1, 8), b1.reshape(1, 8), W2, b2.reshape(1, 4))
